# Initial kernel scaffold; baseline (speedup 1.0000x reference)
import jax
import jax.numpy as jnp
from jax.experimental import pallas as pl


def kernel(output, label_batch, prob_threshold):
    raise NotImplementedError("write your pallas kernel here")


# speedup vs baseline: 2.0907x; 2.0907x over previous
import jax
import jax.numpy as jnp
from jax.experimental import pallas as pl
from jax.experimental.pallas import tpu as pltpu

_B, _A, _C = 16, 25200, 85
_NLAB = 20
_BLK = 8400
_K = _A // _BLK


def _loss_kernel(thr_ref, x_ref, lab_ref, num_ref, den_ref):
    k = pl.program_id(1)
    thr = thr_ref[0]
    x = x_ref[0]
    lab = lab_ref[0]

    eye = (jax.lax.broadcasted_iota(jnp.int32, (_C, _C), 0)
           == jax.lax.broadcasted_iota(jnp.int32, (_C, _C), 1)
           ).astype(jnp.float32)
    t = jax.lax.dot_general(eye, x, (((1,), (1,)), ((), ())),
                            preferred_element_type=jnp.float32)

    obj = t[4:5, :]
    keep = obj >= thr

    e = jnp.exp(t * obj)
    rowmask = (jax.lax.broadcasted_iota(jnp.int32, (_C, 1), 0)
               >= 5).astype(jnp.float32)
    esum = jnp.sum(e * rowmask, axis=0, keepdims=True)
    lse = jnp.log(esum)

    cx, cy, w, h = t[0:1, :], t[1:2, :], t[2:3, :], t[3:4, :]
    px1 = cx - w * 0.5
    py1 = cy - h * 0.5
    px2 = cx + w * 0.5
    py2 = cy + h * 0.5
    parea = (px2 - px1) * (py2 - py1)

    lcx, lcy, lw, lh = lab[:, 1:2], lab[:, 2:3], lab[:, 3:4], lab[:, 4:5]
    lx1 = jnp.clip(lcx - lw * 0.5, 0.0, 1.0)
    ly1 = jnp.clip(lcy - lh * 0.5, 0.0, 1.0)
    lx2 = jnp.clip(lcx + lw * 0.5, 0.0, 1.0)
    ly2 = jnp.clip(lcy + lh * 0.5, 0.0, 1.0)
    larea = (lx2 - lx1) * (ly2 - ly1)

    ltx = jnp.maximum(px1, lx1)
    lty = jnp.maximum(py1, ly1)
    rbx = jnp.minimum(px2, lx2)
    rby = jnp.minimum(py2, ly2)
    iw = jnp.maximum(rbx - ltx, 0.0)
    ih = jnp.maximum(rby - lty, 0.0)
    inter = iw * ih
    iou = inter / (parea + larea - inter)
    mask = (iou >= 0.5) & keep
    maskf = mask.astype(jnp.float32)

    cid = lab[:, 0:1].astype(jnp.int32) + 5
    onehot = (jax.lax.broadcasted_iota(jnp.int32, (_NLAB, _C), 1)
              == cid).astype(jnp.float32)
    g = jax.lax.dot_general(onehot, t, (((1,), (0,)), ((), ())),
                            preferred_element_type=jnp.float32)

    num_p = jnp.sum(maskf * (lse - g * obj))
    den_p = jnp.sum(maskf)

    @pl.when(k == 0)
    def _():
        num_ref[...] = jnp.zeros_like(num_ref)
        den_ref[...] = jnp.zeros_like(den_ref)

    num_ref[...] += num_p
    den_ref[...] += den_p


@jax.jit
def _run(output, label_batch, thr):
    num, den = pl.pallas_call(
        _loss_kernel,
        grid_spec=pltpu.PrefetchScalarGridSpec(
            num_scalar_prefetch=1,
            grid=(_B, _K),
            in_specs=[
                pl.BlockSpec((1, _BLK, _C), lambda i, k, *_: (i, k, 0)),
                pl.BlockSpec((1, _NLAB, 5), lambda i, k, *_: (i, 0, 0)),
            ],
            out_specs=[
                pl.BlockSpec((1, 1, 128), lambda i, k, *_: (i, 0, 0)),
                pl.BlockSpec((1, 1, 128), lambda i, k, *_: (i, 0, 0)),
            ],
        ),
        out_shape=[
            jax.ShapeDtypeStruct((_B, 1, 128), jnp.float32),
            jax.ShapeDtypeStruct((_B, 1, 128), jnp.float32),
        ],
        compiler_params=pltpu.CompilerParams(
            dimension_semantics=("parallel", "arbitrary"),
        ),
    )(thr, output, label_batch)
    return num[:, 0, 0], den[:, 0, 0]


def kernel(output, label_batch, prob_threshold):
    thr = jnp.reshape(jnp.asarray(prob_threshold, jnp.float32), (1,))
    num, den = _run(output, label_batch, thr)
    return (num / den)[None, :]


# speedup vs baseline: 2.1458x; 1.0263x over previous
import jax
import jax.numpy as jnp
from jax.experimental import pallas as pl
from jax.experimental.pallas import tpu as pltpu

_B, _A, _C = 16, 25200, 85
_NLAB = 20
_BLK = 12600
_K = _A // _BLK


def _loss_kernel(thr_ref, x_ref, lab_ref, num_ref, den_ref):
    k = pl.program_id(1)
    thr = thr_ref[0]
    x = x_ref[0]
    lab = lab_ref[0]

    eye = (jax.lax.broadcasted_iota(jnp.int32, (_C, _C), 0)
           == jax.lax.broadcasted_iota(jnp.int32, (_C, _C), 1)
           ).astype(jnp.float32)
    t = jax.lax.dot_general(eye, x, (((1,), (1,)), ((), ())),
                            preferred_element_type=jnp.float32)

    obj = t[4:5, :]
    keep = obj >= thr

    e = jnp.exp(t * obj)
    rowmask = (jax.lax.broadcasted_iota(jnp.int32, (_C, 1), 0)
               >= 5).astype(jnp.float32)
    esum = jnp.sum(e * rowmask, axis=0, keepdims=True)
    lse = jnp.log(esum)

    cx, cy, w, h = t[0:1, :], t[1:2, :], t[2:3, :], t[3:4, :]
    px1 = cx - w * 0.5
    py1 = cy - h * 0.5
    px2 = cx + w * 0.5
    py2 = cy + h * 0.5
    parea = (px2 - px1) * (py2 - py1)

    lcx, lcy, lw, lh = lab[:, 1:2], lab[:, 2:3], lab[:, 3:4], lab[:, 4:5]
    lx1 = jnp.clip(lcx - lw * 0.5, 0.0, 1.0)
    ly1 = jnp.clip(lcy - lh * 0.5, 0.0, 1.0)
    lx2 = jnp.clip(lcx + lw * 0.5, 0.0, 1.0)
    ly2 = jnp.clip(lcy + lh * 0.5, 0.0, 1.0)
    larea = (lx2 - lx1) * (ly2 - ly1)

    ltx = jnp.maximum(px1, lx1)
    lty = jnp.maximum(py1, ly1)
    rbx = jnp.minimum(px2, lx2)
    rby = jnp.minimum(py2, ly2)
    iw = jnp.maximum(rbx - ltx, 0.0)
    ih = jnp.maximum(rby - lty, 0.0)
    inter = iw * ih
    iou = inter / (parea + larea - inter)
    mask = (iou >= 0.5) & keep
    maskf = mask.astype(jnp.float32)

    cid = lab[:, 0:1].astype(jnp.int32) + 5
    onehot = (jax.lax.broadcasted_iota(jnp.int32, (_NLAB, _C), 1)
              == cid).astype(jnp.float32)
    g = jax.lax.dot_general(onehot, t, (((1,), (0,)), ((), ())),
                            preferred_element_type=jnp.float32)

    num_p = jnp.sum(maskf * (lse - g * obj))
    den_p = jnp.sum(maskf)

    @pl.when(k == 0)
    def _():
        num_ref[...] = jnp.zeros_like(num_ref)
        den_ref[...] = jnp.zeros_like(den_ref)

    num_ref[...] += num_p
    den_ref[...] += den_p


@jax.jit
def _run(output, label_batch, thr):
    num, den = pl.pallas_call(
        _loss_kernel,
        grid_spec=pltpu.PrefetchScalarGridSpec(
            num_scalar_prefetch=1,
            grid=(_B, _K),
            in_specs=[
                pl.BlockSpec((1, _BLK, _C), lambda i, k, *_: (i, k, 0)),
                pl.BlockSpec((1, _NLAB, 5), lambda i, k, *_: (i, 0, 0)),
            ],
            out_specs=[
                pl.BlockSpec((1, 1, 128), lambda i, k, *_: (i, 0, 0)),
                pl.BlockSpec((1, 1, 128), lambda i, k, *_: (i, 0, 0)),
            ],
        ),
        out_shape=[
            jax.ShapeDtypeStruct((_B, 1, 128), jnp.float32),
            jax.ShapeDtypeStruct((_B, 1, 128), jnp.float32),
        ],
        compiler_params=pltpu.CompilerParams(
            dimension_semantics=("parallel", "arbitrary"),
        ),
    )(thr, output, label_batch)
    return num[:, 0, 0], den[:, 0, 0]


def kernel(output, label_batch, prob_threshold):
    thr = jnp.reshape(jnp.asarray(prob_threshold, jnp.float32), (1,))
    num, den = _run(output, label_batch, thr)
    return (num / den)[None, :]
